# baseline (speedup 1.0000x reference)
import jax
import jax.numpy as jnp
from jax.experimental import pallas as pl

_HINGE = 0.01
_BQ = 256
_BK = 16384
_NL = 128
_K5 = 5


def _loss_kernel(src_ref, tgt_ref, tsq_ref, out_ref):
    a = src_ref[...]
    a2 = jnp.sum(a * a, axis=1, keepdims=True)
    aneg = a * -2.0
    k_total = tgt_ref.shape[0]
    inf = jnp.float32(jnp.inf)
    init = tuple(jnp.full((_BQ, _NL), inf, jnp.float32) for _ in range(_K5))

    def chunk_body(c, state):
        b = tgt_ref[pl.ds(c * _BK, _BK), :]
        b2 = tsq_ref[:, pl.ds(c * _BK, _BK)]
        ab2 = jax.lax.dot_general(
            aneg, b, (((1,), (1,)), ((), ())),
            preferred_element_type=jnp.float32)
        d2 = ab2 + b2

        for j in range(_BK // _NL):
            e = d2[:, j * _NL:(j + 1) * _NL]
            new = []
            for m in state:
                new.append(jnp.minimum(m, e))
                e = jnp.maximum(m, e)
            state = tuple(new)
        return state

    state = init
    for c in range(k_total // _BK):
        state = chunk_body(c, state)

    cand = jnp.concatenate(state, axis=1)
    width = _K5 * _NL
    col = jax.lax.broadcasted_iota(jnp.int32, (_BQ, width), 1)
    vals = []
    for _ in range(_K5):
        rowmin = jnp.min(cand, axis=1, keepdims=True)
        sel = jnp.where(cand == rowmin, col, width)
        first = jnp.min(sel, axis=1, keepdims=True)
        cand = jnp.where(col == first, inf, cand)
        vals.append(rowmin)
    d2_top = jnp.concatenate(vals, axis=1) + a2
    d = jnp.sqrt(jnp.maximum(d2_top, 0.0))
    out_ref[...] = jnp.maximum(d - _HINGE, 0.0)


@jax.jit
def _hinged_bottom5(source, target, tsq):
    q, d = source.shape
    k = target.shape[0]
    return pl.pallas_call(
        _loss_kernel,
        grid=(q // _BQ,),
        in_specs=[
            pl.BlockSpec((_BQ, d), lambda i: (i, 0)),
            pl.BlockSpec((k, d), lambda i: (0, 0)),
            pl.BlockSpec((1, k), lambda i: (0, 0)),
        ],
        out_specs=pl.BlockSpec((_BQ, _K5), lambda i: (i, 0)),
        out_shape=jax.ShapeDtypeStruct((q, _K5), jnp.float32),
    )(source, target, tsq)


def kernel(source, target, top_k):
    tsq = jnp.sum(target * target, axis=1)[None, :]
    vals = _hinged_bottom5(source, target, tsq)
    loss = jnp.mean(vals)
    return loss + 0.0 * jnp.asarray(top_k, dtype=loss.dtype)


# speedup vs baseline: 1.0730x; 1.0730x over previous
import jax
import jax.numpy as jnp
from jax.experimental import pallas as pl

_HINGE = 0.01
_BQ = 512
_BK = 8192
_NL = 128
_K5 = 5


def _loss_kernel(src_ref, tgt_ref, tsq_ref, out_ref):
    a = src_ref[...]
    a2 = jnp.sum(a * a, axis=1, keepdims=True)
    aneg = a * -2.0
    k_total = tgt_ref.shape[0]
    inf = jnp.float32(jnp.inf)
    init = tuple(jnp.full((_BQ, _NL), inf, jnp.float32) for _ in range(_K5))

    def chunk_body(c, state):
        b = tgt_ref[pl.ds(c * _BK, _BK), :]
        b2 = tsq_ref[:, pl.ds(c * _BK, _BK)]
        ab2 = jax.lax.dot_general(
            aneg, b, (((1,), (1,)), ((), ())),
            preferred_element_type=jnp.float32)
        d2 = ab2 + b2

        for j in range(_BK // _NL):
            e = d2[:, j * _NL:(j + 1) * _NL]
            new = []
            for m in state:
                new.append(jnp.minimum(m, e))
                e = jnp.maximum(m, e)
            state = tuple(new)
        return state

    state = init
    for c in range(k_total // _BK):
        state = chunk_body(c, state)

    cand = jnp.concatenate(state, axis=1)
    width = _K5 * _NL
    col = jax.lax.broadcasted_iota(jnp.int32, (_BQ, width), 1)
    vals = []
    for _ in range(_K5):
        rowmin = jnp.min(cand, axis=1, keepdims=True)
        sel = jnp.where(cand == rowmin, col, width)
        first = jnp.min(sel, axis=1, keepdims=True)
        cand = jnp.where(col == first, inf, cand)
        vals.append(rowmin)
    d2_top = jnp.concatenate(vals, axis=1) + a2
    d = jnp.sqrt(jnp.maximum(d2_top, 0.0))
    out_ref[...] = jnp.maximum(d - _HINGE, 0.0)


@jax.jit
def _hinged_bottom5(source, target, tsq):
    q, d = source.shape
    k = target.shape[0]
    return pl.pallas_call(
        _loss_kernel,
        grid=(q // _BQ,),
        in_specs=[
            pl.BlockSpec((_BQ, d), lambda i: (i, 0)),
            pl.BlockSpec((k, d), lambda i: (0, 0)),
            pl.BlockSpec((1, k), lambda i: (0, 0)),
        ],
        out_specs=pl.BlockSpec((_BQ, _K5), lambda i: (i, 0)),
        out_shape=jax.ShapeDtypeStruct((q, _K5), jnp.float32),
    )(source, target, tsq)


def kernel(source, target, top_k):
    tsq = jnp.sum(target * target, axis=1)[None, :]
    vals = _hinged_bottom5(source, target, tsq)
    loss = jnp.mean(vals)
    return loss + 0.0 * jnp.asarray(top_k, dtype=loss.dtype)


# speedup vs baseline: 1.1569x; 1.0782x over previous
import jax
import jax.numpy as jnp
from jax.experimental import pallas as pl

_HINGE = 0.01
_BQ = 1024
_BK = 4096
_NL = 128
_K5 = 5


def _loss_kernel(src_ref, tgt_ref, tsq_ref, out_ref):
    a = src_ref[...]
    a2 = jnp.sum(a * a, axis=1, keepdims=True)
    aneg = a * -2.0
    k_total = tgt_ref.shape[0]
    inf = jnp.float32(jnp.inf)
    init = tuple(jnp.full((_BQ, _NL), inf, jnp.float32) for _ in range(_K5))

    def chunk_body(c, state):
        b = tgt_ref[pl.ds(c * _BK, _BK), :]
        b2 = tsq_ref[:, pl.ds(c * _BK, _BK)]
        ab2 = jax.lax.dot_general(
            aneg, b, (((1,), (1,)), ((), ())),
            preferred_element_type=jnp.float32)
        d2 = ab2 + b2

        for j in range(_BK // _NL):
            e = d2[:, j * _NL:(j + 1) * _NL]
            new = []
            for m in state:
                new.append(jnp.minimum(m, e))
                e = jnp.maximum(m, e)
            state = tuple(new)
        return state

    state = init
    for c in range(k_total // _BK):
        state = chunk_body(c, state)

    cand = jnp.concatenate(state, axis=1)
    width = _K5 * _NL
    col = jax.lax.broadcasted_iota(jnp.int32, (_BQ, width), 1)
    vals = []
    for _ in range(_K5):
        rowmin = jnp.min(cand, axis=1, keepdims=True)
        sel = jnp.where(cand == rowmin, col, width)
        first = jnp.min(sel, axis=1, keepdims=True)
        cand = jnp.where(col == first, inf, cand)
        vals.append(rowmin)
    d2_top = jnp.concatenate(vals, axis=1) + a2
    d = jnp.sqrt(jnp.maximum(d2_top, 0.0))
    out_ref[...] = jnp.maximum(d - _HINGE, 0.0)


@jax.jit
def _hinged_bottom5(source, target, tsq):
    q, d = source.shape
    k = target.shape[0]
    return pl.pallas_call(
        _loss_kernel,
        grid=(q // _BQ,),
        in_specs=[
            pl.BlockSpec((_BQ, d), lambda i: (i, 0)),
            pl.BlockSpec((k, d), lambda i: (0, 0)),
            pl.BlockSpec((1, k), lambda i: (0, 0)),
        ],
        out_specs=pl.BlockSpec((_BQ, _K5), lambda i: (i, 0)),
        out_shape=jax.ShapeDtypeStruct((q, _K5), jnp.float32),
    )(source, target, tsq)


def kernel(source, target, top_k):
    tsq = jnp.sum(target * target, axis=1)[None, :]
    vals = _hinged_bottom5(source, target, tsq)
    loss = jnp.mean(vals)
    return loss + 0.0 * jnp.asarray(top_k, dtype=loss.dtype)
